# baseline (speedup 1.0000x reference)
import jax
import jax.numpy as jnp
from jax import lax
from jax.experimental import pallas as pl
from jax.experimental.pallas import tpu as pltpu
from jax.experimental.pallas import tpu_sc as plsc

N = 10000
E = 320000
TPAD = 20096
D = 128
L = 2
KMAX = 2

NC = 2
NS = 16
NW = NC * NS
C = 128
TRIPS = 80
EP = NW * TRIPS * C
EPW = E // NW
RPT = N // NS
DPT = 2560
DPAD = NS * DPT
TRASH = DPAD - 1
NBUF = 2
EH = EP // NW // 2
NCHUNK = E // C
CFLOOR = NCHUNK // NW
CEXTRA = NCHUNK % NW

_mesh = plsc.VectorSubcoreMesh(core_axis_name="c", subcore_axis_name="s")
_sc_params = pltpu.CompilerParams(needs_layout_passes=False)


def _wid():
    return lax.axis_index("c") * NS + lax.axis_index("s")


def _deg_body(src_hbm, dst_hbm, attr_hbm, out_hbm,
              deg_sh, src_v, dst_v, attr_v, gidx2_v, sidx2_v,
              ones_v, deg_v):
    cid = lax.axis_index("c")
    sid = lax.axis_index("s")
    wid = cid * NS + sid
    z16 = jnp.zeros((16,), jnp.float32)
    o16 = jnp.ones((16,), jnp.float32)
    i16 = lax.iota(jnp.int32, 16)
    t16 = jnp.full((16,), TRASH, jnp.int32)
    epp = TRIPS * C
    base0 = wid * epp

    def zrow(i, _):
        deg_v[pl.ds(i * 16, 16)] = z16
        return 0
    lax.fori_loop(0, DPT // 16, zrow, 0)
    pltpu.sync_copy(deg_v, deg_sh.at[pl.ds(sid * DPT, DPT)])

    def orow(i, _):
        ones_v[pl.ds(i * 16, 16)] = o16
        return 0
    lax.fori_loop(0, C // 16, orow, 0)
    plsc.subcore_barrier()

    pltpu.sync_copy(src_hbm.at[pl.ds(base0, epp)], src_v)
    pltpu.sync_copy(dst_hbm.at[pl.ds(base0, epp)], dst_v)
    pltpu.sync_copy(attr_hbm.at[pl.ds(base0, epp)], attr_v)

    def irow(r, _):
        for jj in range(C // 16):
            sl = pl.ds(r * C + jj * 16, 16)
            a16 = attr_v[sl] - 1
            real = (base0 + r * C + jj * 16 + i16) < E
            gidx2_v[r, pl.ds(jj * 16, 16)] = jnp.where(
                real, a16 * N + src_v[sl], t16)
            sidx2_v[r, pl.ds(jj * 16, 16)] = jnp.where(
                real, 2 * N + a16 * N + dst_v[sl], t16)
        pltpu.sync_copy(ones_v, deg_sh.at[gidx2_v.at[r]], add=True)
        pltpu.sync_copy(ones_v, deg_sh.at[sidx2_v.at[r]], add=True)
        return 0
    lax.fori_loop(0, TRIPS, irow, 0)
    plsc.subcore_barrier()

    pltpu.sync_copy(deg_sh.at[pl.ds(sid * DPT, DPT)], deg_v)
    pltpu.sync_copy(deg_v, out_hbm.at[cid, sid])


_deg_call = pl.kernel(
    _deg_body,
    out_type=jax.ShapeDtypeStruct((NC, NS, DPT), jnp.float32),
    mesh=_mesh,
    compiler_params=_sc_params,
    scratch_types=[
        pltpu.VMEM_SHARED((DPAD,), jnp.float32),
        pltpu.VMEM((TRIPS * C,), jnp.int32),
        pltpu.VMEM((TRIPS * C,), jnp.int32),
        pltpu.VMEM((TRIPS * C,), jnp.int32),
        pltpu.VMEM((TRIPS, C), jnp.int32),
        pltpu.VMEM((TRIPS, C), jnp.int32),
        pltpu.VMEM((C,), jnp.float32),
        pltpu.VMEM((DPT,), jnp.float32),
    ],
)


def _tables_body(degp_ref, alpha_ref, rsrc_ref, rdst_ref):
    deg = degp_ref[0:1, :] + degp_ref[1:2, :]
    r = lax.rsqrt(jnp.maximum(deg, 1.0))
    rsrc_ref[...] = r[:, 0:2 * N]
    al = alpha_ref[...]
    m = jnp.max(al, axis=1, keepdims=True)
    ex = jnp.exp(al - m)
    a = ex / jnp.sum(ex, axis=1, keepdims=True)
    rd = r[:, 2 * N:4 * N]
    for t in range(L):
        for k in range(KMAX):
            rdst_ref[t:t + 1, k * N:(k + 1) * N] = (
                a[t:t + 1, k:k + 1] * rd[:, k * N:(k + 1) * N])


_tables_call = pl.pallas_call(
    _tables_body,
    out_shape=(jax.ShapeDtypeStruct((1, 2 * N), jnp.float32),
               jax.ShapeDtypeStruct((L, 2 * N), jnp.float32)),
)


def _w_body(src_hbm, dst_hbm, attr_hbm, rsrc_hbm, rdst_hbm, w0_hbm, w1_hbm,
            rsrc_v, rdst_v, src_v, dst_v, attr_v, w0_v, w1_v):
    wid = _wid()
    pltpu.sync_copy(rsrc_hbm, rsrc_v)
    pltpu.sync_copy(rdst_hbm, rdst_v)

    def chunk(g, _):
        base = (wid + g * NW) * C
        pltpu.sync_copy(src_hbm.at[pl.ds(base, C)], src_v)
        pltpu.sync_copy(dst_hbm.at[pl.ds(base, C)], dst_v)
        pltpu.sync_copy(attr_hbm.at[pl.ds(base, C)], attr_v)
        for j in range(C // 16):
            sl = pl.ds(j * 16, 16)
            a16 = attr_v[sl] - 1
            g16 = a16 * N + src_v[sl]
            s16 = a16 * N + dst_v[sl]
            rs = plsc.load_gather(rsrc_v, [g16])
            rd0 = plsc.load_gather(rdst_v, [s16])
            rd1 = plsc.load_gather(rdst_v, [s16 + 2 * N])
            w0_v[sl] = rs * rd0
            w1_v[sl] = rs * rd1
        pltpu.sync_copy(w0_v, w0_hbm.at[pl.ds(base, C)])
        pltpu.sync_copy(w1_v, w1_hbm.at[pl.ds(base, C)])
        return 0
    trips = CFLOOR + jnp.where(wid < CEXTRA, 1, 0)
    lax.fori_loop(0, trips, chunk, 0)


_w_call = pl.kernel(
    _w_body,
    out_type=(jax.ShapeDtypeStruct((E,), jnp.float32),
              jax.ShapeDtypeStruct((E,), jnp.float32)),
    mesh=_mesh,
    compiler_params=_sc_params,
    scratch_types=[
        pltpu.VMEM((2 * N,), jnp.float32),
        pltpu.VMEM((2 * L * N,), jnp.float32),
        pltpu.VMEM((C,), jnp.int32),
        pltpu.VMEM((C,), jnp.int32),
        pltpu.VMEM((C,), jnp.int32),
        pltpu.VMEM((C,), jnp.float32),
        pltpu.VMEM((C,), jnp.float32),
    ],
)


def _agg_body(h_hbm, src3_hbm, dst3_hbm, w_hbm, out_hbm,
              acc_sh, sidxr, didxr, wb0, wb1,
              r0, r1, g0, g1, i0, i1):
    cid = lax.axis_index("c")
    sid = lax.axis_index("s")
    wid = cid * NS + sid
    z16 = jnp.zeros((16,), jnp.float32)
    rbufs = [r0, r1]
    wbufs = [wb0, wb1]
    sgs = [g0, g1]
    sis = [i0, i1]
    T2 = 2 * TRIPS
    CC = C // 2
    row0 = wid * T2

    def zrow(i, _):
        for dd in range(D // 16):
            r0[i, pl.ds(dd * 16, 16)] = z16
        return 0
    lax.fori_loop(0, CC, zrow, 0)

    def zcp(q, _):
        pltpu.sync_copy(r0.at[pl.ds(0, 5)],
                        acc_sh.at[pl.ds(sid * RPT + q * 5, 5)])
        return 0
    lax.fori_loop(0, RPT // 5, zcp, 0)
    plsc.subcore_barrier()

    pltpu.async_copy(src3_hbm.at[row0], sidxr.at[0], i0)
    pltpu.async_copy(dst3_hbm.at[row0], didxr.at[0], i0)

    def step(i, b):
        @pl.when(i < T2)
        def _fire():
            i4 = lax.rem(i, 4)
            pltpu.make_async_copy(src3_hbm.at[row0], sidxr.at[0],
                                  sis[b]).wait()
            pltpu.make_async_copy(dst3_hbm.at[row0], didxr.at[0],
                                  sis[b]).wait()

            pltpu.async_copy(h_hbm.at[sidxr.at[i4]], rbufs[b], sgs[b])
            pltpu.async_copy(w_hbm.at[row0 + i], wbufs[b], sgs[b])

            @pl.when(i + 1 < T2)
            def _prefetch_idx():
                i4n = lax.rem(i + 1, 4)
                pltpu.async_copy(src3_hbm.at[row0 + i + 1], sidxr.at[i4n],
                                 sis[1 - b])
                pltpu.async_copy(dst3_hbm.at[row0 + i + 1], didxr.at[i4n],
                                 sis[1 - b])

        @pl.when((i >= 1) & (i < T2 + 1))
        def _process():
            bp = (b + 1) % NBUF
            g = i - 1
            pltpu.make_async_copy(h_hbm.at[pl.ds(0, CC)],
                                  rbufs[bp], sgs[bp]).wait()
            pltpu.make_async_copy(w_hbm.at[row0], wbufs[bp],
                                  sgs[bp]).wait()

            def sbody(e, _):
                wv = plsc.load_gather(
                    wbufs[bp], [e + jnp.zeros((16,), jnp.int32)])
                for dd in range(D // 16):
                    sl = pl.ds(dd * 16, 16)
                    rbufs[bp][e, sl] = rbufs[bp][e, sl] * wv
                return 0
            lax.fori_loop(0, CC, sbody, 0)
            pltpu.sync_copy(rbufs[bp], acc_sh.at[didxr.at[lax.rem(g, 4)]],
                            add=True)

    def ring(j, _):
        for b in range(NBUF):
            step(j * NBUF + b, b)
        return 0
    lax.fori_loop(0, (T2 + 1 + NBUF - 1) // NBUF, ring, 0)

    plsc.subcore_barrier()
    m8 = sid % 8
    start = pl.multiple_of(sid * RPT - m8, 8)
    pltpu.sync_copy(acc_sh.at[pl.ds(start, 624)],
                    out_hbm.at[cid, pl.ds(start, 624)])

    @pl.when(m8 == 7)
    def _dump_tail():
        pltpu.sync_copy(acc_sh.at[pl.ds(start + 624, 8)],
                        out_hbm.at[cid, pl.ds(start + 624, 8)])


_agg_call = pl.kernel(
    _agg_body,
    out_type=jax.ShapeDtypeStruct((NC, N, D), jnp.float32),
    mesh=_mesh,
    compiler_params=_sc_params,
    scratch_types=[
        pltpu.VMEM_SHARED((N + 8, D), jnp.float32),
        pltpu.VMEM((4, C // 2), jnp.int32),
        pltpu.VMEM((4, C // 2), jnp.int32),
        pltpu.VMEM((C // 2,), jnp.float32),
        pltpu.VMEM((C // 2,), jnp.float32),
        pltpu.VMEM((C // 2, D), jnp.float32),
        pltpu.VMEM((C // 2, D), jnp.float32),
        pltpu.SemaphoreType.DMA,
        pltpu.SemaphoreType.DMA,
        pltpu.SemaphoreType.DMA,
        pltpu.SemaphoreType.DMA,
    ],
)


def _make_fuse(has_parts, has_mm, block):
    grid = N // block

    def body(*refs):
        refs = list(refs)
        x_ref = refs.pop(0)
        if has_parts:
            p0_ref = refs.pop(0)
            p1_ref = refs.pop(0)
        if has_mm:
            w_ref = refs.pop(0)
            b_ref = refs.pop(0)
        xv = x_ref[...]
        if has_parts:
            xv = xv + jnp.maximum(p0_ref[...] + p1_ref[...], 0.0)
            n2 = jnp.sum(xv * xv, axis=1, keepdims=True)
            xv = xv / jnp.maximum(jnp.sqrt(n2), 1e-12)
            xo_ref = refs.pop(0)
            xo_ref[...] = xv
        if has_mm:
            h_ref = refs.pop(0)
            h_ref[...] = (jnp.dot(xv, w_ref[...],
                                  preferred_element_type=jnp.float32)
                          + b_ref[...])

    row_spec = pl.BlockSpec((block, D), lambda i: (i, 0))
    in_specs = [row_spec]
    if has_parts:
        in_specs += [row_spec, row_spec]
    if has_mm:
        in_specs += [pl.BlockSpec((D, D), lambda i: (0, 0)),
                     pl.BlockSpec((1, D), lambda i: (0, 0))]
    out_shapes = []
    out_specs = []
    if has_parts:
        out_shapes.append(jax.ShapeDtypeStruct((N, D), jnp.float32))
        out_specs.append(row_spec)
    if has_mm:
        out_shapes.append(jax.ShapeDtypeStruct((N, D), jnp.float32))
        out_specs.append(row_spec)
    return pl.pallas_call(
        body,
        grid=(grid,),
        in_specs=in_specs,
        out_specs=out_specs if len(out_specs) > 1 else out_specs[0],
        out_shape=tuple(out_shapes) if len(out_shapes) > 1 else out_shapes[0],
    )


_mm0 = _make_fuse(False, True, 1000)
_fuse_mm = _make_fuse(True, True, 1000)
_fuse_last = _make_fuse(True, False, 1000)


def kernel(x, edge_index, edge_attr, W, b, alpha):
    src = edge_index[0]
    dst = edge_index[1]
    attr = edge_attr.astype(jnp.int32)

    npad = EP - E
    srcp = jnp.concatenate([src, jnp.zeros((npad,), jnp.int32)])
    dstp = jnp.concatenate([dst, jnp.full((npad,), N, jnp.int32)])
    attrp = jnp.concatenate([attr, jnp.ones((npad,), jnp.int32)])
    src3 = srcp.reshape(2 * (EP // C), C // 2)
    dst3 = dstp.reshape(2 * (EP // C), C // 2)

    degp = _deg_call(srcp, dstp, attrp).reshape(NC, DPAD)[:, :4 * N]
    rsrc, rdst = _tables_call(degp, alpha)
    w0, w1 = _w_call(src, dst, attr, rsrc.reshape(2 * N),
                     rdst.reshape(2 * L * N))
    zw = jnp.zeros((EP - E,), jnp.float32)
    w0r = jnp.concatenate([w0, zw]).reshape(2 * (EP // C), C // 2)
    w1r = jnp.concatenate([w1, zw]).reshape(2 * (EP // C), C // 2)
    h0 = _mm0(x, W[0], b[0].reshape(1, D))
    parts0 = _agg_call(h0, src3, dst3, w0r)
    x1, h1 = _fuse_mm(x, parts0[0], parts0[1], W[1], b[1].reshape(1, D))
    parts1 = _agg_call(h1, src3, dst3, w1r)
    x2 = _fuse_last(x1, parts1[0], parts1[1])
    return x2


# speedup vs baseline: 1.0191x; 1.0191x over previous
import jax
import jax.numpy as jnp
from jax import lax
from jax.experimental import pallas as pl
from jax.experimental.pallas import tpu as pltpu
from jax.experimental.pallas import tpu_sc as plsc

N = 10000
E = 320000
TPAD = 20096
D = 128
L = 2
KMAX = 2

NC = 2
NS = 16
NW = NC * NS
C = 128
TRIPS = 80
EP = NW * TRIPS * C
EPW = E // NW
RPT = N // NS
DPT = 2560
DPAD = NS * DPT
TRASH = DPAD - 1
NBUF = 2
EH = EP // NW // 2
NCHUNK = E // C
CFLOOR = NCHUNK // NW
CEXTRA = NCHUNK % NW

_mesh = plsc.VectorSubcoreMesh(core_axis_name="c", subcore_axis_name="s")
_sc_params = pltpu.CompilerParams(needs_layout_passes=False)


def _wid():
    return lax.axis_index("c") * NS + lax.axis_index("s")


def _deg_body(src_hbm, dst_hbm, attr_hbm, out_hbm,
              deg_sh, src_v, dst_v, attr_v, gidx2_v, sidx2_v,
              ones_v, deg_v):
    cid = lax.axis_index("c")
    sid = lax.axis_index("s")
    wid = cid * NS + sid
    z16 = jnp.zeros((16,), jnp.float32)
    o16 = jnp.ones((16,), jnp.float32)
    i16 = lax.iota(jnp.int32, 16)
    t16 = jnp.full((16,), TRASH, jnp.int32)
    epp = TRIPS * C
    base0 = wid * epp

    def zrow(i, _):
        deg_v[pl.ds(i * 16, 16)] = z16
        return 0
    lax.fori_loop(0, DPT // 16, zrow, 0)
    pltpu.sync_copy(deg_v, deg_sh.at[pl.ds(sid * DPT, DPT)])

    def orow(i, _):
        ones_v[pl.ds(i * 16, 16)] = o16
        return 0
    lax.fori_loop(0, C // 16, orow, 0)
    plsc.subcore_barrier()

    pltpu.sync_copy(src_hbm.at[pl.ds(base0, epp)], src_v)
    pltpu.sync_copy(dst_hbm.at[pl.ds(base0, epp)], dst_v)
    pltpu.sync_copy(attr_hbm.at[pl.ds(base0, epp)], attr_v)

    def irow(r, _):
        for jj in range(C // 16):
            sl = pl.ds(r * C + jj * 16, 16)
            a16 = attr_v[sl] - 1
            real = (base0 + r * C + jj * 16 + i16) < E
            gidx2_v[r, pl.ds(jj * 16, 16)] = jnp.where(
                real, a16 * N + src_v[sl], t16)
            sidx2_v[r, pl.ds(jj * 16, 16)] = jnp.where(
                real, 2 * N + a16 * N + dst_v[sl], t16)
        pltpu.sync_copy(ones_v, deg_sh.at[gidx2_v.at[r]], add=True)
        pltpu.sync_copy(ones_v, deg_sh.at[sidx2_v.at[r]], add=True)
        return 0
    lax.fori_loop(0, TRIPS, irow, 0)
    plsc.subcore_barrier()

    pltpu.sync_copy(deg_sh.at[pl.ds(sid * DPT, DPT)], deg_v)
    pltpu.sync_copy(deg_v, out_hbm.at[cid, sid])


_deg_call = pl.kernel(
    _deg_body,
    out_type=jax.ShapeDtypeStruct((NC, NS, DPT), jnp.float32),
    mesh=_mesh,
    compiler_params=_sc_params,
    scratch_types=[
        pltpu.VMEM_SHARED((DPAD,), jnp.float32),
        pltpu.VMEM((TRIPS * C,), jnp.int32),
        pltpu.VMEM((TRIPS * C,), jnp.int32),
        pltpu.VMEM((TRIPS * C,), jnp.int32),
        pltpu.VMEM((TRIPS, C), jnp.int32),
        pltpu.VMEM((TRIPS, C), jnp.int32),
        pltpu.VMEM((C,), jnp.float32),
        pltpu.VMEM((DPT,), jnp.float32),
    ],
)


def _tables_body(degp_ref, alpha_ref, rsrc_ref, rdst_ref):
    deg = degp_ref[0:1, :] + degp_ref[1:2, :]
    r = lax.rsqrt(jnp.maximum(deg, 1.0))
    rsrc_ref[...] = r[:, 0:2 * N]
    al = alpha_ref[...]
    m = jnp.max(al, axis=1, keepdims=True)
    ex = jnp.exp(al - m)
    a = ex / jnp.sum(ex, axis=1, keepdims=True)
    rd = r[:, 2 * N:4 * N]
    for t in range(L):
        for k in range(KMAX):
            rdst_ref[t:t + 1, k * N:(k + 1) * N] = (
                a[t:t + 1, k:k + 1] * rd[:, k * N:(k + 1) * N])


_tables_call = pl.pallas_call(
    _tables_body,
    out_shape=(jax.ShapeDtypeStruct((1, 2 * N), jnp.float32),
               jax.ShapeDtypeStruct((L, 2 * N), jnp.float32)),
)


def _w_body(src_hbm, dst_hbm, attr_hbm, rsrc_hbm, rdst_hbm, w0_hbm, w1_hbm,
            rsrc_v, rdst_v, src_v, dst_v, attr_v, w0_v, w1_v):
    wid = _wid()
    pltpu.sync_copy(rsrc_hbm, rsrc_v)
    pltpu.sync_copy(rdst_hbm, rdst_v)

    def chunk(g, _):
        base = (wid + g * NW) * C
        pltpu.sync_copy(src_hbm.at[pl.ds(base, C)], src_v)
        pltpu.sync_copy(dst_hbm.at[pl.ds(base, C)], dst_v)
        pltpu.sync_copy(attr_hbm.at[pl.ds(base, C)], attr_v)
        for j in range(C // 16):
            sl = pl.ds(j * 16, 16)
            a16 = attr_v[sl] - 1
            g16 = a16 * N + src_v[sl]
            s16 = a16 * N + dst_v[sl]
            rs = plsc.load_gather(rsrc_v, [g16])
            rd0 = plsc.load_gather(rdst_v, [s16])
            rd1 = plsc.load_gather(rdst_v, [s16 + 2 * N])
            w0_v[sl] = rs * rd0
            w1_v[sl] = rs * rd1
        pltpu.sync_copy(w0_v, w0_hbm.at[pl.ds(base, C)])
        pltpu.sync_copy(w1_v, w1_hbm.at[pl.ds(base, C)])
        return 0
    trips = CFLOOR + jnp.where(wid < CEXTRA, 1, 0)
    lax.fori_loop(0, trips, chunk, 0)


_w_call = pl.kernel(
    _w_body,
    out_type=(jax.ShapeDtypeStruct((E,), jnp.float32),
              jax.ShapeDtypeStruct((E,), jnp.float32)),
    mesh=_mesh,
    compiler_params=_sc_params,
    scratch_types=[
        pltpu.VMEM((2 * N,), jnp.float32),
        pltpu.VMEM((2 * L * N,), jnp.float32),
        pltpu.VMEM((C,), jnp.int32),
        pltpu.VMEM((C,), jnp.int32),
        pltpu.VMEM((C,), jnp.int32),
        pltpu.VMEM((C,), jnp.float32),
        pltpu.VMEM((C,), jnp.float32),
    ],
)


def _agg_body(h_hbm, src3_hbm, dst3_hbm, w_hbm, out_hbm,
              acc_sh, sidxr, didxr, wb0, wb1,
              r0, r1, g0, g1, i0, i1):
    cid = lax.axis_index("c")
    sid = lax.axis_index("s")
    wid = cid * NS + sid
    z16 = jnp.zeros((16,), jnp.float32)
    rbufs = [r0, r1]
    wbufs = [wb0, wb1]
    sgs = [g0, g1]
    sis = [i0, i1]
    T2 = 2 * TRIPS
    CC = C // 2
    row0 = wid * T2

    def zrow(i, _):
        for dd in range(D // 16):
            r0[i, pl.ds(dd * 16, 16)] = z16
        return 0
    lax.fori_loop(0, CC, zrow, 0)

    def zcp(q, _):
        pltpu.sync_copy(r0.at[pl.ds(0, 5)],
                        acc_sh.at[pl.ds(sid * RPT + q * 5, 5)])
        return 0
    lax.fori_loop(0, RPT // 5, zcp, 0)
    plsc.subcore_barrier()

    pltpu.async_copy(src3_hbm.at[row0], sidxr.at[0], i0)
    pltpu.async_copy(dst3_hbm.at[row0], didxr.at[0], i0)

    def step(i, b):
        @pl.when(i < T2)
        def _fire():
            i4 = lax.rem(i, 4)
            pltpu.make_async_copy(src3_hbm.at[row0], sidxr.at[0],
                                  sis[b]).wait()
            pltpu.make_async_copy(dst3_hbm.at[row0], didxr.at[0],
                                  sis[b]).wait()

            pltpu.async_copy(h_hbm.at[sidxr.at[i4]], rbufs[b], sgs[b])
            pltpu.async_copy(w_hbm.at[row0 + i], wbufs[b], sgs[b])

            @pl.when(i + 1 < T2)
            def _prefetch_idx():
                i4n = lax.rem(i + 1, 4)
                pltpu.async_copy(src3_hbm.at[row0 + i + 1], sidxr.at[i4n],
                                 sis[1 - b])
                pltpu.async_copy(dst3_hbm.at[row0 + i + 1], didxr.at[i4n],
                                 sis[1 - b])

        @pl.when((i >= 1) & (i < T2 + 1))
        def _process():
            bp = (b + 1) % NBUF
            g = i - 1
            pltpu.make_async_copy(h_hbm.at[pl.ds(0, CC)],
                                  rbufs[bp], sgs[bp]).wait()
            pltpu.make_async_copy(w_hbm.at[row0], wbufs[bp],
                                  sgs[bp]).wait()

            def sbody(h2, _):
                e = h2 * 2
                zi16 = jnp.zeros((16,), jnp.int32)
                wv0 = plsc.load_gather(wbufs[bp], [e + zi16])
                wv1 = plsc.load_gather(wbufs[bp], [e + 1 + zi16])
                for dd in range(D // 16):
                    sl = pl.ds(dd * 16, 16)
                    rbufs[bp][e, sl] = rbufs[bp][e, sl] * wv0
                    rbufs[bp][e + 1, sl] = rbufs[bp][e + 1, sl] * wv1
                return 0
            lax.fori_loop(0, CC // 2, sbody, 0)
            pltpu.sync_copy(rbufs[bp], acc_sh.at[didxr.at[lax.rem(g, 4)]],
                            add=True)

    def ring(j, _):
        for b in range(NBUF):
            step(j * NBUF + b, b)
        return 0
    lax.fori_loop(0, (T2 + 1 + NBUF - 1) // NBUF, ring, 0)

    plsc.subcore_barrier()
    m8 = sid % 8
    start = pl.multiple_of(sid * RPT - m8, 8)
    pltpu.sync_copy(acc_sh.at[pl.ds(start, 624)],
                    out_hbm.at[cid, pl.ds(start, 624)])

    @pl.when(m8 == 7)
    def _dump_tail():
        pltpu.sync_copy(acc_sh.at[pl.ds(start + 624, 8)],
                        out_hbm.at[cid, pl.ds(start + 624, 8)])


_agg_call = pl.kernel(
    _agg_body,
    out_type=jax.ShapeDtypeStruct((NC, N, D), jnp.float32),
    mesh=_mesh,
    compiler_params=_sc_params,
    scratch_types=[
        pltpu.VMEM_SHARED((N + 8, D), jnp.float32),
        pltpu.VMEM((4, C // 2), jnp.int32),
        pltpu.VMEM((4, C // 2), jnp.int32),
        pltpu.VMEM((C // 2,), jnp.float32),
        pltpu.VMEM((C // 2,), jnp.float32),
        pltpu.VMEM((C // 2, D), jnp.float32),
        pltpu.VMEM((C // 2, D), jnp.float32),
        pltpu.SemaphoreType.DMA,
        pltpu.SemaphoreType.DMA,
        pltpu.SemaphoreType.DMA,
        pltpu.SemaphoreType.DMA,
    ],
)


def _make_fuse(has_parts, has_mm, block):
    grid = N // block

    def body(*refs):
        refs = list(refs)
        x_ref = refs.pop(0)
        if has_parts:
            p0_ref = refs.pop(0)
            p1_ref = refs.pop(0)
        if has_mm:
            w_ref = refs.pop(0)
            b_ref = refs.pop(0)
        xv = x_ref[...]
        if has_parts:
            xv = xv + jnp.maximum(p0_ref[...] + p1_ref[...], 0.0)
            n2 = jnp.sum(xv * xv, axis=1, keepdims=True)
            xv = xv / jnp.maximum(jnp.sqrt(n2), 1e-12)
            xo_ref = refs.pop(0)
            xo_ref[...] = xv
        if has_mm:
            h_ref = refs.pop(0)
            h_ref[...] = (jnp.dot(xv, w_ref[...],
                                  preferred_element_type=jnp.float32)
                          + b_ref[...])

    row_spec = pl.BlockSpec((block, D), lambda i: (i, 0))
    in_specs = [row_spec]
    if has_parts:
        in_specs += [row_spec, row_spec]
    if has_mm:
        in_specs += [pl.BlockSpec((D, D), lambda i: (0, 0)),
                     pl.BlockSpec((1, D), lambda i: (0, 0))]
    out_shapes = []
    out_specs = []
    if has_parts:
        out_shapes.append(jax.ShapeDtypeStruct((N, D), jnp.float32))
        out_specs.append(row_spec)
    if has_mm:
        out_shapes.append(jax.ShapeDtypeStruct((N, D), jnp.float32))
        out_specs.append(row_spec)
    return pl.pallas_call(
        body,
        grid=(grid,),
        in_specs=in_specs,
        out_specs=out_specs if len(out_specs) > 1 else out_specs[0],
        out_shape=tuple(out_shapes) if len(out_shapes) > 1 else out_shapes[0],
    )


_mm0 = _make_fuse(False, True, 1000)
_fuse_mm = _make_fuse(True, True, 1000)
_fuse_last = _make_fuse(True, False, 1000)


def kernel(x, edge_index, edge_attr, W, b, alpha):
    src = edge_index[0]
    dst = edge_index[1]
    attr = edge_attr.astype(jnp.int32)

    npad = EP - E
    srcp = jnp.concatenate([src, jnp.zeros((npad,), jnp.int32)])
    dstp = jnp.concatenate([dst, jnp.full((npad,), N, jnp.int32)])
    attrp = jnp.concatenate([attr, jnp.ones((npad,), jnp.int32)])
    src3 = srcp.reshape(2 * (EP // C), C // 2)
    dst3 = dstp.reshape(2 * (EP // C), C // 2)

    degp = _deg_call(srcp, dstp, attrp).reshape(NC, DPAD)[:, :4 * N]
    rsrc, rdst = _tables_call(degp, alpha)
    w0, w1 = _w_call(src, dst, attr, rsrc.reshape(2 * N),
                     rdst.reshape(2 * L * N))
    zw = jnp.zeros((EP - E,), jnp.float32)
    w0r = jnp.concatenate([w0, zw]).reshape(2 * (EP // C), C // 2)
    w1r = jnp.concatenate([w1, zw]).reshape(2 * (EP // C), C // 2)
    h0 = _mm0(x, W[0], b[0].reshape(1, D))
    parts0 = _agg_call(h0, src3, dst3, w0r)
    x1, h1 = _fuse_mm(x, parts0[0], parts0[1], W[1], b[1].reshape(1, D))
    parts1 = _agg_call(h1, src3, dst3, w1r)
    x2 = _fuse_last(x1, parts1[0], parts1[1])
    return x2


# speedup vs baseline: 1.2691x; 1.2453x over previous
import jax
import jax.numpy as jnp
from jax import lax
from jax.experimental import pallas as pl
from jax.experimental.pallas import tpu as pltpu
from jax.experimental.pallas import tpu_sc as plsc

N = 10000
E = 320000
TPAD = 20096
D = 128
L = 2
KMAX = 2

NC = 2
NS = 16
NW = NC * NS
C = 128
TRIPS = 80
EP = NW * TRIPS * C
EPW = E // NW
RPT = N // NS
DPT = 2560
DPAD = NS * DPT
TRASH = DPAD - 1
NBUF = 2
EH = EP // NW // 2
NCHUNK = E // C
CFLOOR = NCHUNK // NW
CEXTRA = NCHUNK % NW

_mesh = plsc.VectorSubcoreMesh(core_axis_name="c", subcore_axis_name="s")
_sc_params = pltpu.CompilerParams(needs_layout_passes=False)


def _wid():
    return lax.axis_index("c") * NS + lax.axis_index("s")


def _deg_body(src_hbm, dst_hbm, attr_hbm, out_hbm,
              deg_sh, src_v, dst_v, attr_v, gidx2_v, sidx2_v,
              ones_v, deg_v):
    cid = lax.axis_index("c")
    sid = lax.axis_index("s")
    wid = cid * NS + sid
    z16 = jnp.zeros((16,), jnp.float32)
    o16 = jnp.ones((16,), jnp.float32)
    i16 = lax.iota(jnp.int32, 16)
    t16 = jnp.full((16,), TRASH, jnp.int32)
    epp = TRIPS * C
    base0 = wid * epp

    def zrow(i, _):
        deg_v[pl.ds(i * 16, 16)] = z16
        return 0
    lax.fori_loop(0, DPT // 16, zrow, 0)
    pltpu.sync_copy(deg_v, deg_sh.at[pl.ds(sid * DPT, DPT)])

    def orow(i, _):
        ones_v[pl.ds(i * 16, 16)] = o16
        return 0
    lax.fori_loop(0, C // 16, orow, 0)
    plsc.subcore_barrier()

    pltpu.sync_copy(src_hbm.at[pl.ds(base0, epp)], src_v)
    pltpu.sync_copy(dst_hbm.at[pl.ds(base0, epp)], dst_v)
    pltpu.sync_copy(attr_hbm.at[pl.ds(base0, epp)], attr_v)

    def irow(r, _):
        for jj in range(C // 16):
            sl = pl.ds(r * C + jj * 16, 16)
            a16 = attr_v[sl] - 1
            real = (base0 + r * C + jj * 16 + i16) < E
            gidx2_v[r, pl.ds(jj * 16, 16)] = jnp.where(
                real, a16 * N + src_v[sl], t16)
            sidx2_v[r, pl.ds(jj * 16, 16)] = jnp.where(
                real, 2 * N + a16 * N + dst_v[sl], t16)
        pltpu.sync_copy(ones_v, deg_sh.at[gidx2_v.at[r]], add=True)
        pltpu.sync_copy(ones_v, deg_sh.at[sidx2_v.at[r]], add=True)
        return 0
    lax.fori_loop(0, TRIPS, irow, 0)
    plsc.subcore_barrier()

    pltpu.sync_copy(deg_sh.at[pl.ds(sid * DPT, DPT)], deg_v)
    pltpu.sync_copy(deg_v, out_hbm.at[cid, sid])


_deg_call = pl.kernel(
    _deg_body,
    out_type=jax.ShapeDtypeStruct((NC, NS, DPT), jnp.float32),
    mesh=_mesh,
    compiler_params=_sc_params,
    scratch_types=[
        pltpu.VMEM_SHARED((DPAD,), jnp.float32),
        pltpu.VMEM((TRIPS * C,), jnp.int32),
        pltpu.VMEM((TRIPS * C,), jnp.int32),
        pltpu.VMEM((TRIPS * C,), jnp.int32),
        pltpu.VMEM((TRIPS, C), jnp.int32),
        pltpu.VMEM((TRIPS, C), jnp.int32),
        pltpu.VMEM((C,), jnp.float32),
        pltpu.VMEM((DPT,), jnp.float32),
    ],
)


def _tables_body(degp_ref, alpha_ref, rsrc_ref, rdst_ref):
    deg = degp_ref[0:1, :] + degp_ref[1:2, :]
    r = lax.rsqrt(jnp.maximum(deg, 1.0))
    rsrc_ref[...] = r[:, 0:2 * N]
    al = alpha_ref[...]
    m = jnp.max(al, axis=1, keepdims=True)
    ex = jnp.exp(al - m)
    a = ex / jnp.sum(ex, axis=1, keepdims=True)
    rd = r[:, 2 * N:4 * N]
    for t in range(L):
        for k in range(KMAX):
            rdst_ref[t:t + 1, k * N:(k + 1) * N] = (
                a[t:t + 1, k:k + 1] * rd[:, k * N:(k + 1) * N])


_tables_call = pl.pallas_call(
    _tables_body,
    out_shape=(jax.ShapeDtypeStruct((1, 2 * N), jnp.float32),
               jax.ShapeDtypeStruct((L, 2 * N), jnp.float32)),
)


def _w_body(src_hbm, dst_hbm, attr_hbm, rsrc_hbm, rdst_hbm, w0_hbm, w1_hbm,
            rsrc_v, rdst_v, src_v, dst_v, attr_v, w0_v, w1_v):
    wid = _wid()
    pltpu.sync_copy(rsrc_hbm, rsrc_v)
    pltpu.sync_copy(rdst_hbm, rdst_v)

    def chunk(g, _):
        base = (wid + g * NW) * C
        pltpu.sync_copy(src_hbm.at[pl.ds(base, C)], src_v)
        pltpu.sync_copy(dst_hbm.at[pl.ds(base, C)], dst_v)
        pltpu.sync_copy(attr_hbm.at[pl.ds(base, C)], attr_v)
        for j in range(C // 16):
            sl = pl.ds(j * 16, 16)
            a16 = attr_v[sl] - 1
            g16 = a16 * N + src_v[sl]
            s16 = a16 * N + dst_v[sl]
            rs = plsc.load_gather(rsrc_v, [g16])
            rd0 = plsc.load_gather(rdst_v, [s16])
            rd1 = plsc.load_gather(rdst_v, [s16 + 2 * N])
            w0_v[sl] = rs * rd0
            w1_v[sl] = rs * rd1
        pltpu.sync_copy(w0_v, w0_hbm.at[pl.ds(base, C)])
        pltpu.sync_copy(w1_v, w1_hbm.at[pl.ds(base, C)])
        return 0
    trips = CFLOOR + jnp.where(wid < CEXTRA, 1, 0)
    lax.fori_loop(0, trips, chunk, 0)


_w_call = pl.kernel(
    _w_body,
    out_type=(jax.ShapeDtypeStruct((E,), jnp.float32),
              jax.ShapeDtypeStruct((E,), jnp.float32)),
    mesh=_mesh,
    compiler_params=_sc_params,
    scratch_types=[
        pltpu.VMEM((2 * N,), jnp.float32),
        pltpu.VMEM((2 * L * N,), jnp.float32),
        pltpu.VMEM((C,), jnp.int32),
        pltpu.VMEM((C,), jnp.int32),
        pltpu.VMEM((C,), jnp.int32),
        pltpu.VMEM((C,), jnp.float32),
        pltpu.VMEM((C,), jnp.float32),
    ],
)


def _agg_body(h_hbm, src_hbm, dst_hbm, w_hbm, out_hbm,
              acc_sh, sidx_v, didx_v, w_v, rows_v, sem):
    cid = lax.axis_index("c")
    sid = lax.axis_index("s")
    wid = cid * NS + sid
    z16 = jnp.zeros((16,), jnp.float32)

    def zrow(i, _):
        for dd in range(D // 16):
            rows_v[i, pl.ds(dd * 16, 16)] = z16
        return 0
    lax.fori_loop(0, C, zrow, 0)

    def zcp(q, _):
        pltpu.sync_copy(rows_v.at[pl.ds(0, 5)],
                        acc_sh.at[pl.ds(sid * RPT + q * 5, 5)])
        return 0
    lax.fori_loop(0, RPT // 5, zcp, 0)
    plsc.subcore_barrier()

    def chunk(g, _):
        base = (wid + g * NW) * C
        pltpu.sync_copy(src_hbm.at[pl.ds(base, C)], sidx_v)
        pltpu.sync_copy(dst_hbm.at[pl.ds(base, C)], didx_v)
        pltpu.sync_copy(w_hbm.at[pl.ds(base, C)], w_v)
        pltpu.async_copy(h_hbm.at[sidx_v], rows_v, sem).wait()

        def sbody(h2, _):
            e = h2 * 2
            zi16 = jnp.zeros((16,), jnp.int32)
            wv0 = plsc.load_gather(w_v, [e + zi16])
            wv1 = plsc.load_gather(w_v, [e + 1 + zi16])
            for dd in range(D // 16):
                sl = pl.ds(dd * 16, 16)
                rows_v[e, sl] = rows_v[e, sl] * wv0
                rows_v[e + 1, sl] = rows_v[e + 1, sl] * wv1
            return 0
        lax.fori_loop(0, C // 2, sbody, 0)
        pltpu.sync_copy(rows_v, acc_sh.at[didx_v], add=True)
        return 0
    trips = CFLOOR + jnp.where(wid < CEXTRA, 1, 0)
    lax.fori_loop(0, trips, chunk, 0)

    plsc.subcore_barrier()
    m8 = sid % 8
    start = pl.multiple_of(sid * RPT - m8, 8)
    pltpu.sync_copy(acc_sh.at[pl.ds(start, 624)],
                    out_hbm.at[cid, pl.ds(start, 624)])

    @pl.when(m8 == 7)
    def _dump_tail():
        pltpu.sync_copy(acc_sh.at[pl.ds(start + 624, 8)],
                        out_hbm.at[cid, pl.ds(start + 624, 8)])


_agg_call = pl.kernel(
    _agg_body,
    out_type=jax.ShapeDtypeStruct((NC, N, D), jnp.float32),
    mesh=_mesh,
    compiler_params=_sc_params,
    scratch_types=[
        pltpu.VMEM_SHARED((N, D), jnp.float32),
        pltpu.VMEM((C,), jnp.int32),
        pltpu.VMEM((C,), jnp.int32),
        pltpu.VMEM((C,), jnp.float32),
        pltpu.VMEM((C, D), jnp.float32),
        pltpu.SemaphoreType.DMA,
    ],
)


def _make_fuse(has_parts, has_mm, block):
    grid = N // block

    def body(*refs):
        refs = list(refs)
        x_ref = refs.pop(0)
        if has_parts:
            p0_ref = refs.pop(0)
            p1_ref = refs.pop(0)
        if has_mm:
            w_ref = refs.pop(0)
            b_ref = refs.pop(0)
        xv = x_ref[...]
        if has_parts:
            xv = xv + jnp.maximum(p0_ref[...] + p1_ref[...], 0.0)
            n2 = jnp.sum(xv * xv, axis=1, keepdims=True)
            xv = xv / jnp.maximum(jnp.sqrt(n2), 1e-12)
            xo_ref = refs.pop(0)
            xo_ref[...] = xv
        if has_mm:
            h_ref = refs.pop(0)
            h_ref[...] = (jnp.dot(xv, w_ref[...],
                                  preferred_element_type=jnp.float32)
                          + b_ref[...])

    row_spec = pl.BlockSpec((block, D), lambda i: (i, 0))
    in_specs = [row_spec]
    if has_parts:
        in_specs += [row_spec, row_spec]
    if has_mm:
        in_specs += [pl.BlockSpec((D, D), lambda i: (0, 0)),
                     pl.BlockSpec((1, D), lambda i: (0, 0))]
    out_shapes = []
    out_specs = []
    if has_parts:
        out_shapes.append(jax.ShapeDtypeStruct((N, D), jnp.float32))
        out_specs.append(row_spec)
    if has_mm:
        out_shapes.append(jax.ShapeDtypeStruct((N, D), jnp.float32))
        out_specs.append(row_spec)
    return pl.pallas_call(
        body,
        grid=(grid,),
        in_specs=in_specs,
        out_specs=out_specs if len(out_specs) > 1 else out_specs[0],
        out_shape=tuple(out_shapes) if len(out_shapes) > 1 else out_shapes[0],
    )


_mm0 = _make_fuse(False, True, 1000)
_fuse_mm = _make_fuse(True, True, 1000)
_fuse_last = _make_fuse(True, False, 1000)


def kernel(x, edge_index, edge_attr, W, b, alpha):
    src = edge_index[0]
    dst = edge_index[1]
    attr = edge_attr.astype(jnp.int32)

    npad = EP - E
    srcp = jnp.concatenate([src, jnp.zeros((npad,), jnp.int32)])
    dstp = jnp.concatenate([dst, jnp.full((npad,), N, jnp.int32)])
    attrp = jnp.concatenate([attr, jnp.ones((npad,), jnp.int32)])

    degp = _deg_call(srcp, dstp, attrp).reshape(NC, DPAD)[:, :4 * N]
    rsrc, rdst = _tables_call(degp, alpha)
    w0, w1 = _w_call(src, dst, attr, rsrc.reshape(2 * N),
                     rdst.reshape(2 * L * N))
    h0 = _mm0(x, W[0], b[0].reshape(1, D))
    parts0 = _agg_call(h0, src, dst, w0)
    x1, h1 = _fuse_mm(x, parts0[0], parts0[1], W[1], b[1].reshape(1, D))
    parts1 = _agg_call(h1, src, dst, w1)
    x2 = _fuse_last(x1, parts1[0], parts1[1])
    return x2


# speedup vs baseline: 1.6303x; 1.2846x over previous
import jax
import jax.numpy as jnp
from jax import lax
from jax.experimental import pallas as pl
from jax.experimental.pallas import tpu as pltpu
from jax.experimental.pallas import tpu_sc as plsc

N = 10000
E = 320000
TPAD = 20096
D = 128
L = 2
KMAX = 2

NC = 2
NS = 16
NW = NC * NS
C = 128
TRIPS = 80
EP = NW * TRIPS * C
EPW = E // NW
RPT = N // NS
DPT = 2560
DPAD = NS * DPT
TRASH = DPAD - 1
NBUF = 2
EH = EP // NW // 2
NCHUNK = E // C
CFLOOR = NCHUNK // NW
CEXTRA = NCHUNK % NW

_mesh = plsc.VectorSubcoreMesh(core_axis_name="c", subcore_axis_name="s")
_sc_params = pltpu.CompilerParams(needs_layout_passes=False)


def _wid():
    return lax.axis_index("c") * NS + lax.axis_index("s")


def _deg_body(src_hbm, dst_hbm, attr_hbm, out_hbm,
              deg_sh, src_v, dst_v, attr_v, gidx2_v, sidx2_v,
              ones_v, deg_v):
    cid = lax.axis_index("c")
    sid = lax.axis_index("s")
    wid = cid * NS + sid
    z16 = jnp.zeros((16,), jnp.float32)
    o16 = jnp.ones((16,), jnp.float32)
    i16 = lax.iota(jnp.int32, 16)
    t16 = jnp.full((16,), TRASH, jnp.int32)
    epp = TRIPS * C
    base0 = wid * epp

    def zrow(i, _):
        deg_v[pl.ds(i * 16, 16)] = z16
        return 0
    lax.fori_loop(0, DPT // 16, zrow, 0)
    pltpu.sync_copy(deg_v, deg_sh.at[pl.ds(sid * DPT, DPT)])

    def orow(i, _):
        ones_v[pl.ds(i * 16, 16)] = o16
        return 0
    lax.fori_loop(0, C // 16, orow, 0)
    plsc.subcore_barrier()

    pltpu.sync_copy(src_hbm.at[pl.ds(base0, epp)], src_v)
    pltpu.sync_copy(dst_hbm.at[pl.ds(base0, epp)], dst_v)
    pltpu.sync_copy(attr_hbm.at[pl.ds(base0, epp)], attr_v)

    def irow(r, _):
        for jj in range(C // 16):
            sl = pl.ds(r * C + jj * 16, 16)
            a16 = attr_v[sl] - 1
            real = (base0 + r * C + jj * 16 + i16) < E
            gidx2_v[r, pl.ds(jj * 16, 16)] = jnp.where(
                real, a16 * N + src_v[sl], t16)
            sidx2_v[r, pl.ds(jj * 16, 16)] = jnp.where(
                real, 2 * N + a16 * N + dst_v[sl], t16)
        pltpu.sync_copy(ones_v, deg_sh.at[gidx2_v.at[r]], add=True)
        pltpu.sync_copy(ones_v, deg_sh.at[sidx2_v.at[r]], add=True)
        return 0
    lax.fori_loop(0, TRIPS, irow, 0)
    plsc.subcore_barrier()

    pltpu.sync_copy(deg_sh.at[pl.ds(sid * DPT, DPT)], deg_v)
    pltpu.sync_copy(deg_v, out_hbm.at[cid, sid])


_deg_call = pl.kernel(
    _deg_body,
    out_type=jax.ShapeDtypeStruct((NC, NS, DPT), jnp.float32),
    mesh=_mesh,
    compiler_params=_sc_params,
    scratch_types=[
        pltpu.VMEM_SHARED((DPAD,), jnp.float32),
        pltpu.VMEM((TRIPS * C,), jnp.int32),
        pltpu.VMEM((TRIPS * C,), jnp.int32),
        pltpu.VMEM((TRIPS * C,), jnp.int32),
        pltpu.VMEM((TRIPS, C), jnp.int32),
        pltpu.VMEM((TRIPS, C), jnp.int32),
        pltpu.VMEM((C,), jnp.float32),
        pltpu.VMEM((DPT,), jnp.float32),
    ],
)


def _tables_body(degp_ref, alpha_ref, rsrc_ref, rdst_ref):
    deg = degp_ref[0:1, :] + degp_ref[1:2, :]
    r = lax.rsqrt(jnp.maximum(deg, 1.0))
    rsrc_ref[...] = r[:, 0:2 * N]
    al = alpha_ref[...]
    m = jnp.max(al, axis=1, keepdims=True)
    ex = jnp.exp(al - m)
    a = ex / jnp.sum(ex, axis=1, keepdims=True)
    rd = r[:, 2 * N:4 * N]
    for t in range(L):
        for k in range(KMAX):
            rdst_ref[t:t + 1, k * N:(k + 1) * N] = (
                a[t:t + 1, k:k + 1] * rd[:, k * N:(k + 1) * N])


_tables_call = pl.pallas_call(
    _tables_body,
    out_shape=(jax.ShapeDtypeStruct((1, 2 * N), jnp.float32),
               jax.ShapeDtypeStruct((L, 2 * N), jnp.float32)),
)


def _w_body(src_hbm, dst_hbm, attr_hbm, rsrc_hbm, rdst_hbm, w0_hbm, w1_hbm,
            rsrc_v, rdst_v, src_v, dst_v, attr_v, w0_v, w1_v):
    wid = _wid()
    pltpu.sync_copy(rsrc_hbm, rsrc_v)
    pltpu.sync_copy(rdst_hbm, rdst_v)

    def chunk(g, _):
        base = (wid + g * NW) * C
        pltpu.sync_copy(src_hbm.at[pl.ds(base, C)], src_v)
        pltpu.sync_copy(dst_hbm.at[pl.ds(base, C)], dst_v)
        pltpu.sync_copy(attr_hbm.at[pl.ds(base, C)], attr_v)
        for j in range(C // 16):
            sl = pl.ds(j * 16, 16)
            a16 = attr_v[sl] - 1
            g16 = a16 * N + src_v[sl]
            s16 = a16 * N + dst_v[sl]
            rs = plsc.load_gather(rsrc_v, [g16])
            rd0 = plsc.load_gather(rdst_v, [s16])
            rd1 = plsc.load_gather(rdst_v, [s16 + 2 * N])
            w0_v[sl] = rs * rd0
            w1_v[sl] = rs * rd1
        pltpu.sync_copy(w0_v, w0_hbm.at[pl.ds(base, C)])
        pltpu.sync_copy(w1_v, w1_hbm.at[pl.ds(base, C)])
        return 0
    trips = CFLOOR + jnp.where(wid < CEXTRA, 1, 0)
    lax.fori_loop(0, trips, chunk, 0)


_w_call = pl.kernel(
    _w_body,
    out_type=(jax.ShapeDtypeStruct((E,), jnp.float32),
              jax.ShapeDtypeStruct((E,), jnp.float32)),
    mesh=_mesh,
    compiler_params=_sc_params,
    scratch_types=[
        pltpu.VMEM((2 * N,), jnp.float32),
        pltpu.VMEM((2 * L * N,), jnp.float32),
        pltpu.VMEM((C,), jnp.int32),
        pltpu.VMEM((C,), jnp.int32),
        pltpu.VMEM((C,), jnp.int32),
        pltpu.VMEM((C,), jnp.float32),
        pltpu.VMEM((C,), jnp.float32),
    ],
)


def _agg_body(h_hbm, src_hbm, dst_hbm, w_hbm, out_hbm,
              acc_sh, sidx_v, didx_v, w_v, rows_v, sem, si0, si1):
    cid = lax.axis_index("c")
    sid = lax.axis_index("s")
    wid = cid * NS + sid
    z16 = jnp.zeros((16,), jnp.float32)

    def zrow(i, _):
        for dd in range(D // 16):
            rows_v[i, pl.ds(dd * 16, 16)] = z16
        return 0
    lax.fori_loop(0, C, zrow, 0)

    def zcp(q, _):
        pltpu.sync_copy(rows_v.at[pl.ds(0, 5)],
                        acc_sh.at[pl.ds(sid * RPT + q * 5, 5)])
        return 0
    lax.fori_loop(0, RPT // 5, zcp, 0)
    plsc.subcore_barrier()

    trips = CFLOOR + jnp.where(wid < CEXTRA, 1, 0)
    sis = [si0, si1]

    base00 = wid * C
    pltpu.async_copy(src_hbm.at[pl.ds(base00, C)], sidx_v.at[0], si0)
    pltpu.async_copy(dst_hbm.at[pl.ds(base00, C)], didx_v.at[0], si0)
    pltpu.async_copy(w_hbm.at[pl.ds(base00, C)], w_v.at[0], si0)

    def chunk(g, _):
        p = lax.rem(g, 2)

        def wait_idx(b):
            pltpu.make_async_copy(src_hbm.at[pl.ds(0, C)],
                                  sidx_v.at[0], sis[b]).wait()
            pltpu.make_async_copy(dst_hbm.at[pl.ds(0, C)],
                                  didx_v.at[0], sis[b]).wait()
            pltpu.make_async_copy(w_hbm.at[pl.ds(0, C)],
                                  w_v.at[0], sis[b]).wait()

        @pl.when(p == 0)
        def _w0():
            wait_idx(0)

        @pl.when(p == 1)
        def _w1():
            wait_idx(1)

        @pl.when(g + 1 < trips)
        def _prefetch():
            nbase = (wid + (g + 1) * NW) * C
            pn = 1 - p

            def fire(b):
                pltpu.async_copy(src_hbm.at[pl.ds(nbase, C)],
                                 sidx_v.at[1 - p], sis[b])
                pltpu.async_copy(dst_hbm.at[pl.ds(nbase, C)],
                                 didx_v.at[1 - p], sis[b])
                pltpu.async_copy(w_hbm.at[pl.ds(nbase, C)],
                                 w_v.at[1 - p], sis[b])

            @pl.when(pn == 0)
            def _f0():
                fire(0)

            @pl.when(pn == 1)
            def _f1():
                fire(1)

        pltpu.async_copy(h_hbm.at[sidx_v.at[p]], rows_v, sem).wait()

        def sbody(h2, _):
            e = h2 * 2
            zi16 = jnp.zeros((16,), jnp.int32)
            wv0 = plsc.load_gather(w_v.at[p], [e + zi16])
            wv1 = plsc.load_gather(w_v.at[p], [e + 1 + zi16])
            for dd in range(D // 16):
                sl = pl.ds(dd * 16, 16)
                rows_v[e, sl] = rows_v[e, sl] * wv0
                rows_v[e + 1, sl] = rows_v[e + 1, sl] * wv1
            return 0
        lax.fori_loop(0, C // 2, sbody, 0)
        pltpu.sync_copy(rows_v, acc_sh.at[didx_v.at[p]], add=True)
        return 0
    lax.fori_loop(0, trips, chunk, 0)

    plsc.subcore_barrier()
    m8 = sid % 8
    start = pl.multiple_of(sid * RPT - m8, 8)
    pltpu.sync_copy(acc_sh.at[pl.ds(start, 624)],
                    out_hbm.at[cid, pl.ds(start, 624)])

    @pl.when(m8 == 7)
    def _dump_tail():
        pltpu.sync_copy(acc_sh.at[pl.ds(start + 624, 8)],
                        out_hbm.at[cid, pl.ds(start + 624, 8)])


_agg_call = pl.kernel(
    _agg_body,
    out_type=jax.ShapeDtypeStruct((NC, N, D), jnp.float32),
    mesh=_mesh,
    compiler_params=_sc_params,
    scratch_types=[
        pltpu.VMEM_SHARED((N, D), jnp.float32),
        pltpu.VMEM((2, C), jnp.int32),
        pltpu.VMEM((2, C), jnp.int32),
        pltpu.VMEM((2, C), jnp.float32),
        pltpu.VMEM((C, D), jnp.float32),
        pltpu.SemaphoreType.DMA,
        pltpu.SemaphoreType.DMA,
        pltpu.SemaphoreType.DMA,
    ],
)


def _make_fuse(has_parts, has_mm, block):
    grid = N // block

    def body(*refs):
        refs = list(refs)
        x_ref = refs.pop(0)
        if has_parts:
            p0_ref = refs.pop(0)
            p1_ref = refs.pop(0)
        if has_mm:
            w_ref = refs.pop(0)
            b_ref = refs.pop(0)
        xv = x_ref[...]
        if has_parts:
            xv = xv + jnp.maximum(p0_ref[...] + p1_ref[...], 0.0)
            n2 = jnp.sum(xv * xv, axis=1, keepdims=True)
            xv = xv / jnp.maximum(jnp.sqrt(n2), 1e-12)
            xo_ref = refs.pop(0)
            xo_ref[...] = xv
        if has_mm:
            h_ref = refs.pop(0)
            h_ref[...] = (jnp.dot(xv, w_ref[...],
                                  preferred_element_type=jnp.float32)
                          + b_ref[...])

    row_spec = pl.BlockSpec((block, D), lambda i: (i, 0))
    in_specs = [row_spec]
    if has_parts:
        in_specs += [row_spec, row_spec]
    if has_mm:
        in_specs += [pl.BlockSpec((D, D), lambda i: (0, 0)),
                     pl.BlockSpec((1, D), lambda i: (0, 0))]
    out_shapes = []
    out_specs = []
    if has_parts:
        out_shapes.append(jax.ShapeDtypeStruct((N, D), jnp.float32))
        out_specs.append(row_spec)
    if has_mm:
        out_shapes.append(jax.ShapeDtypeStruct((N, D), jnp.float32))
        out_specs.append(row_spec)
    return pl.pallas_call(
        body,
        grid=(grid,),
        in_specs=in_specs,
        out_specs=out_specs if len(out_specs) > 1 else out_specs[0],
        out_shape=tuple(out_shapes) if len(out_shapes) > 1 else out_shapes[0],
    )


_mm0 = _make_fuse(False, True, 1000)
_fuse_mm = _make_fuse(True, True, 1000)
_fuse_last = _make_fuse(True, False, 1000)


def kernel(x, edge_index, edge_attr, W, b, alpha):
    src = edge_index[0]
    dst = edge_index[1]
    attr = edge_attr.astype(jnp.int32)

    npad = EP - E
    srcp = jnp.concatenate([src, jnp.zeros((npad,), jnp.int32)])
    dstp = jnp.concatenate([dst, jnp.full((npad,), N, jnp.int32)])
    attrp = jnp.concatenate([attr, jnp.ones((npad,), jnp.int32)])

    degp = _deg_call(srcp, dstp, attrp).reshape(NC, DPAD)[:, :4 * N]
    rsrc, rdst = _tables_call(degp, alpha)
    w0, w1 = _w_call(src, dst, attr, rsrc.reshape(2 * N),
                     rdst.reshape(2 * L * N))
    h0 = _mm0(x, W[0], b[0].reshape(1, D))
    parts0 = _agg_call(h0, src, dst, w0)
    x1, h1 = _fuse_mm(x, parts0[0], parts0[1], W[1], b[1].reshape(1, D))
    parts1 = _agg_call(h1, src, dst, w1)
    x2 = _fuse_last(x1, parts1[0], parts1[1])
    return x2


# speedup vs baseline: 1.8311x; 1.1232x over previous
import jax
import jax.numpy as jnp
from jax import lax
from jax.experimental import pallas as pl
from jax.experimental.pallas import tpu as pltpu
from jax.experimental.pallas import tpu_sc as plsc

N = 10000
E = 320000
TPAD = 20096
D = 128
L = 2
KMAX = 2

NC = 2
NS = 16
NW = NC * NS
C = 128
TRIPS = 80
EP = NW * TRIPS * C
EPW = E // NW
RPT = N // NS
DPT = 2560
DPAD = NS * DPT
TRASH = DPAD - 1
NBUF = 2
EH = EP // NW // 2
NCHUNK = E // C
CFLOOR = NCHUNK // NW
CEXTRA = NCHUNK % NW

_mesh = plsc.VectorSubcoreMesh(core_axis_name="c", subcore_axis_name="s")
_sc_params = pltpu.CompilerParams(needs_layout_passes=False)


def _wid():
    return lax.axis_index("c") * NS + lax.axis_index("s")


def _deg_body(src_hbm, dst_hbm, attr_hbm, out_hbm,
              deg_sh, src_v, dst_v, attr_v, gidx2_v, sidx2_v,
              ones_v, deg_v):
    cid = lax.axis_index("c")
    sid = lax.axis_index("s")
    wid = cid * NS + sid
    z16 = jnp.zeros((16,), jnp.float32)
    o16 = jnp.ones((16,), jnp.float32)
    i16 = lax.iota(jnp.int32, 16)
    t16 = jnp.full((16,), TRASH, jnp.int32)
    epp = TRIPS * C
    base0 = wid * epp

    def zrow(i, _):
        deg_v[pl.ds(i * 16, 16)] = z16
        return 0
    lax.fori_loop(0, DPT // 16, zrow, 0)
    pltpu.sync_copy(deg_v, deg_sh.at[pl.ds(sid * DPT, DPT)])

    def orow(i, _):
        ones_v[pl.ds(i * 16, 16)] = o16
        return 0
    lax.fori_loop(0, C // 16, orow, 0)
    plsc.subcore_barrier()

    pltpu.sync_copy(src_hbm.at[pl.ds(base0, epp)], src_v)
    pltpu.sync_copy(dst_hbm.at[pl.ds(base0, epp)], dst_v)
    pltpu.sync_copy(attr_hbm.at[pl.ds(base0, epp)], attr_v)

    def irow(r, _):
        for jj in range(C // 16):
            sl = pl.ds(r * C + jj * 16, 16)
            a16 = attr_v[sl] - 1
            real = (base0 + r * C + jj * 16 + i16) < E
            gidx2_v[r, pl.ds(jj * 16, 16)] = jnp.where(
                real, a16 * N + src_v[sl], t16)
            sidx2_v[r, pl.ds(jj * 16, 16)] = jnp.where(
                real, 2 * N + a16 * N + dst_v[sl], t16)
        pltpu.sync_copy(ones_v, deg_sh.at[gidx2_v.at[r]], add=True)
        pltpu.sync_copy(ones_v, deg_sh.at[sidx2_v.at[r]], add=True)
        return 0
    lax.fori_loop(0, TRIPS, irow, 0)
    plsc.subcore_barrier()

    pltpu.sync_copy(deg_sh.at[pl.ds(sid * DPT, DPT)], deg_v)
    pltpu.sync_copy(deg_v, out_hbm.at[cid, sid])


_deg_call = pl.kernel(
    _deg_body,
    out_type=jax.ShapeDtypeStruct((NC, NS, DPT), jnp.float32),
    mesh=_mesh,
    compiler_params=_sc_params,
    scratch_types=[
        pltpu.VMEM_SHARED((DPAD,), jnp.float32),
        pltpu.VMEM((TRIPS * C,), jnp.int32),
        pltpu.VMEM((TRIPS * C,), jnp.int32),
        pltpu.VMEM((TRIPS * C,), jnp.int32),
        pltpu.VMEM((TRIPS, C), jnp.int32),
        pltpu.VMEM((TRIPS, C), jnp.int32),
        pltpu.VMEM((C,), jnp.float32),
        pltpu.VMEM((DPT,), jnp.float32),
    ],
)


def _tables_body(degp_ref, alpha_ref, rsrc_ref, rdst_ref):
    deg = degp_ref[0:1, :] + degp_ref[1:2, :]
    r = lax.rsqrt(jnp.maximum(deg, 1.0))
    rsrc_ref[...] = r[:, 0:2 * N]
    al = alpha_ref[...]
    m = jnp.max(al, axis=1, keepdims=True)
    ex = jnp.exp(al - m)
    a = ex / jnp.sum(ex, axis=1, keepdims=True)
    rd = r[:, 2 * N:4 * N]
    for t in range(L):
        for k in range(KMAX):
            rdst_ref[t:t + 1, k * N:(k + 1) * N] = (
                a[t:t + 1, k:k + 1] * rd[:, k * N:(k + 1) * N])


_tables_call = pl.pallas_call(
    _tables_body,
    out_shape=(jax.ShapeDtypeStruct((1, 2 * N), jnp.float32),
               jax.ShapeDtypeStruct((L, 2 * N), jnp.float32)),
)


def _w_body(src_hbm, dst_hbm, attr_hbm, rsrc_hbm, rdst_hbm, w0_hbm, w1_hbm,
            rsrc_v, rdst_v, src_v, dst_v, attr_v, w0_v, w1_v, si0, si1):
    wid = _wid()
    pltpu.sync_copy(rsrc_hbm, rsrc_v)
    pltpu.sync_copy(rdst_hbm, rdst_v)
    trips = CFLOOR + jnp.where(wid < CEXTRA, 1, 0)
    sis = [si0, si1]

    base00 = wid * C
    pltpu.async_copy(src_hbm.at[pl.ds(base00, C)], src_v.at[0], si0)
    pltpu.async_copy(dst_hbm.at[pl.ds(base00, C)], dst_v.at[0], si0)
    pltpu.async_copy(attr_hbm.at[pl.ds(base00, C)], attr_v.at[0], si0)

    def chunk(g, _):
        p = lax.rem(g, 2)
        base = (wid + g * NW) * C

        def wait_idx(bb):
            pltpu.make_async_copy(src_hbm.at[pl.ds(0, C)],
                                  src_v.at[0], sis[bb]).wait()
            pltpu.make_async_copy(dst_hbm.at[pl.ds(0, C)],
                                  dst_v.at[0], sis[bb]).wait()
            pltpu.make_async_copy(attr_hbm.at[pl.ds(0, C)],
                                  attr_v.at[0], sis[bb]).wait()

        @pl.when(p == 0)
        def _w0():
            wait_idx(0)

        @pl.when(p == 1)
        def _w1():
            wait_idx(1)

        @pl.when(g + 1 < trips)
        def _prefetch():
            nbase = (wid + (g + 1) * NW) * C
            pn = 1 - p

            def fire(bb):
                pltpu.async_copy(src_hbm.at[pl.ds(nbase, C)],
                                 src_v.at[1 - p], sis[bb])
                pltpu.async_copy(dst_hbm.at[pl.ds(nbase, C)],
                                 dst_v.at[1 - p], sis[bb])
                pltpu.async_copy(attr_hbm.at[pl.ds(nbase, C)],
                                 attr_v.at[1 - p], sis[bb])

            @pl.when(pn == 0)
            def _f0():
                fire(0)

            @pl.when(pn == 1)
            def _f1():
                fire(1)

        for j in range(C // 16):
            sl = pl.ds(j * 16, 16)
            a16 = attr_v[p, sl] - 1
            g16 = a16 * N + src_v[p, sl]
            s16 = a16 * N + dst_v[p, sl]
            rs = plsc.load_gather(rsrc_v, [g16])
            rd0 = plsc.load_gather(rdst_v, [s16])
            rd1 = plsc.load_gather(rdst_v, [s16 + 2 * N])
            w0_v[sl] = rs * rd0
            w1_v[sl] = rs * rd1
        pltpu.sync_copy(w0_v, w0_hbm.at[pl.ds(base, C)])
        pltpu.sync_copy(w1_v, w1_hbm.at[pl.ds(base, C)])
        return 0
    lax.fori_loop(0, trips, chunk, 0)


_w_call = pl.kernel(
    _w_body,
    out_type=(jax.ShapeDtypeStruct((E,), jnp.float32),
              jax.ShapeDtypeStruct((E,), jnp.float32)),
    mesh=_mesh,
    compiler_params=_sc_params,
    scratch_types=[
        pltpu.VMEM((2 * N,), jnp.float32),
        pltpu.VMEM((2 * L * N,), jnp.float32),
        pltpu.VMEM((2, C), jnp.int32),
        pltpu.VMEM((2, C), jnp.int32),
        pltpu.VMEM((2, C), jnp.int32),
        pltpu.VMEM((C,), jnp.float32),
        pltpu.VMEM((C,), jnp.float32),
        pltpu.SemaphoreType.DMA,
        pltpu.SemaphoreType.DMA,
    ],
)


def _agg_body(h_hbm, src_hbm, dst_hbm, w_hbm, out_hbm,
              acc_sh, sidx_v, didx_v, w_v, rows_v, sem, si0, si1):
    cid = lax.axis_index("c")
    sid = lax.axis_index("s")
    wid = cid * NS + sid
    z16 = jnp.zeros((16,), jnp.float32)

    def zrow(i, _):
        for dd in range(D // 16):
            rows_v[i, pl.ds(dd * 16, 16)] = z16
        return 0
    lax.fori_loop(0, C, zrow, 0)

    def zcp(q, _):
        pltpu.sync_copy(rows_v.at[pl.ds(0, 5)],
                        acc_sh.at[pl.ds(sid * RPT + q * 5, 5)])
        return 0
    lax.fori_loop(0, RPT // 5, zcp, 0)
    plsc.subcore_barrier()

    trips = CFLOOR + jnp.where(wid < CEXTRA, 1, 0)
    sis = [si0, si1]

    base00 = wid * C
    pltpu.async_copy(src_hbm.at[pl.ds(base00, C)], sidx_v.at[0], si0)
    pltpu.async_copy(dst_hbm.at[pl.ds(base00, C)], didx_v.at[0], si0)
    pltpu.async_copy(w_hbm.at[pl.ds(base00, C)], w_v.at[0], si0)

    def chunk(g, _):
        p = lax.rem(g, 2)

        def wait_idx(b):
            pltpu.make_async_copy(src_hbm.at[pl.ds(0, C)],
                                  sidx_v.at[0], sis[b]).wait()
            pltpu.make_async_copy(dst_hbm.at[pl.ds(0, C)],
                                  didx_v.at[0], sis[b]).wait()
            pltpu.make_async_copy(w_hbm.at[pl.ds(0, C)],
                                  w_v.at[0], sis[b]).wait()

        @pl.when(p == 0)
        def _w0():
            wait_idx(0)

        @pl.when(p == 1)
        def _w1():
            wait_idx(1)

        @pl.when(g + 1 < trips)
        def _prefetch():
            nbase = (wid + (g + 1) * NW) * C
            pn = 1 - p

            def fire(b):
                pltpu.async_copy(src_hbm.at[pl.ds(nbase, C)],
                                 sidx_v.at[1 - p], sis[b])
                pltpu.async_copy(dst_hbm.at[pl.ds(nbase, C)],
                                 didx_v.at[1 - p], sis[b])
                pltpu.async_copy(w_hbm.at[pl.ds(nbase, C)],
                                 w_v.at[1 - p], sis[b])

            @pl.when(pn == 0)
            def _f0():
                fire(0)

            @pl.when(pn == 1)
            def _f1():
                fire(1)

        pltpu.async_copy(h_hbm.at[sidx_v.at[p]], rows_v, sem).wait()

        def sbody(h2, _):
            e = h2 * 2
            zi16 = jnp.zeros((16,), jnp.int32)
            wv0 = plsc.load_gather(w_v.at[p], [e + zi16])
            wv1 = plsc.load_gather(w_v.at[p], [e + 1 + zi16])
            for dd in range(D // 16):
                sl = pl.ds(dd * 16, 16)
                rows_v[e, sl] = rows_v[e, sl] * wv0
                rows_v[e + 1, sl] = rows_v[e + 1, sl] * wv1
            return 0
        lax.fori_loop(0, C // 2, sbody, 0)
        pltpu.sync_copy(rows_v, acc_sh.at[didx_v.at[p]], add=True)
        return 0
    lax.fori_loop(0, trips, chunk, 0)

    plsc.subcore_barrier()
    m8 = sid % 8
    start = pl.multiple_of(sid * RPT - m8, 8)
    pltpu.sync_copy(acc_sh.at[pl.ds(start, 624)],
                    out_hbm.at[cid, pl.ds(start, 624)])

    @pl.when(m8 == 7)
    def _dump_tail():
        pltpu.sync_copy(acc_sh.at[pl.ds(start + 624, 8)],
                        out_hbm.at[cid, pl.ds(start + 624, 8)])


_agg_call = pl.kernel(
    _agg_body,
    out_type=jax.ShapeDtypeStruct((NC, N, D), jnp.float32),
    mesh=_mesh,
    compiler_params=_sc_params,
    scratch_types=[
        pltpu.VMEM_SHARED((N, D), jnp.float32),
        pltpu.VMEM((2, C), jnp.int32),
        pltpu.VMEM((2, C), jnp.int32),
        pltpu.VMEM((2, C), jnp.float32),
        pltpu.VMEM((C, D), jnp.float32),
        pltpu.SemaphoreType.DMA,
        pltpu.SemaphoreType.DMA,
        pltpu.SemaphoreType.DMA,
    ],
)


def _make_fuse(has_parts, has_mm, block):
    grid = N // block

    def body(*refs):
        refs = list(refs)
        x_ref = refs.pop(0)
        if has_parts:
            p0_ref = refs.pop(0)
            p1_ref = refs.pop(0)
        if has_mm:
            w_ref = refs.pop(0)
            b_ref = refs.pop(0)
        xv = x_ref[...]
        if has_parts:
            xv = xv + jnp.maximum(p0_ref[...] + p1_ref[...], 0.0)
            n2 = jnp.sum(xv * xv, axis=1, keepdims=True)
            xv = xv / jnp.maximum(jnp.sqrt(n2), 1e-12)
            xo_ref = refs.pop(0)
            xo_ref[...] = xv
        if has_mm:
            h_ref = refs.pop(0)
            h_ref[...] = (jnp.dot(xv, w_ref[...],
                                  preferred_element_type=jnp.float32)
                          + b_ref[...])

    row_spec = pl.BlockSpec((block, D), lambda i: (i, 0))
    in_specs = [row_spec]
    if has_parts:
        in_specs += [row_spec, row_spec]
    if has_mm:
        in_specs += [pl.BlockSpec((D, D), lambda i: (0, 0)),
                     pl.BlockSpec((1, D), lambda i: (0, 0))]
    out_shapes = []
    out_specs = []
    if has_parts:
        out_shapes.append(jax.ShapeDtypeStruct((N, D), jnp.float32))
        out_specs.append(row_spec)
    if has_mm:
        out_shapes.append(jax.ShapeDtypeStruct((N, D), jnp.float32))
        out_specs.append(row_spec)
    return pl.pallas_call(
        body,
        grid=(grid,),
        in_specs=in_specs,
        out_specs=out_specs if len(out_specs) > 1 else out_specs[0],
        out_shape=tuple(out_shapes) if len(out_shapes) > 1 else out_shapes[0],
    )


_mm0 = _make_fuse(False, True, 1000)
_fuse_mm = _make_fuse(True, True, 1000)
_fuse_last = _make_fuse(True, False, 1000)


def kernel(x, edge_index, edge_attr, W, b, alpha):
    src = edge_index[0]
    dst = edge_index[1]
    attr = edge_attr.astype(jnp.int32)

    npad = EP - E
    srcp = jnp.concatenate([src, jnp.zeros((npad,), jnp.int32)])
    dstp = jnp.concatenate([dst, jnp.full((npad,), N, jnp.int32)])
    attrp = jnp.concatenate([attr, jnp.ones((npad,), jnp.int32)])

    degp = _deg_call(srcp, dstp, attrp).reshape(NC, DPAD)[:, :4 * N]
    rsrc, rdst = _tables_call(degp, alpha)
    w0, w1 = _w_call(src, dst, attr, rsrc.reshape(2 * N),
                     rdst.reshape(2 * L * N))
    h0 = _mm0(x, W[0], b[0].reshape(1, D))
    parts0 = _agg_call(h0, src, dst, w0)
    x1, h1 = _fuse_mm(x, parts0[0], parts0[1], W[1], b[1].reshape(1, D))
    parts1 = _agg_call(h1, src, dst, w1)
    x2 = _fuse_last(x1, parts1[0], parts1[1])
    return x2
